# baseline (speedup 1.0000x reference)
import functools

import jax
import jax.numpy as jnp
from jax import lax
from jax.experimental import pallas as pl
from jax.experimental.pallas import tpu as pltpu
from jax.experimental.pallas import tpu_sc as plsc

VOCAB = 1000
VPAD = 1024
EMBED = 128
B = 1024
L = 50
BT = B * L

NUM_CORES = 2
NUM_SUBCORES = 16
NW = NUM_CORES * NUM_SUBCORES
PER_W = BT // NW
CHUNK = 80
NCHUNK = PER_W // CHUNK


def _gram_body(w_ref, wp_ref, b_ref, g_ref):
    w = w_ref[...]
    wp = wp_ref[...]
    g = lax.dot_general(
        w, wp,
        dimension_numbers=(((1,), (1,)), ((), ())),
        preferred_element_type=jnp.float32,
    )
    g_ref[...] = g + b_ref[...]


def _gram(W, Wp, b2d):
    return pl.pallas_call(
        _gram_body,
        out_shape=jax.ShapeDtypeStruct((VOCAB, b2d.shape[1]), jnp.float32),
    )(W, Wp, b2d)


_sc_mesh = plsc.VectorSubcoreMesh(core_axis_name="c", subcore_axis_name="s")


@functools.partial(
    pl.kernel,
    mesh=_sc_mesh,
    out_type=jax.ShapeDtypeStruct((BT, VOCAB), jnp.float32),
    scratch_types=[
        pltpu.VMEM((PER_W,), jnp.int32),
        pltpu.VMEM((CHUNK, VOCAB), jnp.float32),
        pltpu.SemaphoreType.DMA,
    ],
    compiler_params=pltpu.CompilerParams(use_tc_tiling_on_sc=False),
)
def _gather(table_hbm, idx_hbm, out_hbm, idx_v, buf_v, sem):
    wid = lax.axis_index("s") * NUM_CORES + lax.axis_index("c")
    base = wid * PER_W
    pltpu.sync_copy(idx_hbm.at[pl.ds(base, PER_W)], idx_v)

    def body(i, carry):
        rows = idx_v.at[pl.ds(i * CHUNK, CHUNK)]
        pltpu.async_copy(table_hbm.at[rows], buf_v, sem).wait()
        pltpu.sync_copy(buf_v, out_hbm.at[pl.ds(base + i * CHUNK, CHUNK)])
        return carry

    lax.fori_loop(0, NCHUNK, body, 0)


def kernel(input_ids, W, b):
    ids = input_ids.reshape(BT).astype(jnp.int32)
    table = _gram(W, W, b.reshape(1, VOCAB))
    out = _gather(table, ids)
    return out.reshape(B, L, VOCAB)
